# initial kernel scaffold (unmeasured)
import jax
import jax.numpy as jnp
from jax import lax
from jax.experimental import pallas as pl
from jax.experimental.pallas import tpu as pltpu


def kernel(
    u,
):
    def body(*refs):
        pass

    out_shape = jax.ShapeDtypeStruct(..., jnp.float32)
    return pl.pallas_call(body, out_shape=out_shape)(...)


# baseline (device time: 13356 ns/iter reference)
import jax
import jax.numpy as jnp
from jax import lax
from jax.experimental import pallas as pl
from jax.experimental.pallas import tpu as pltpu

NX, NY, NZ = 2, 2, 4
S = 32


def kernel(u):
    def body(u_ref, out_ref, halo_x, halo_y, halo_z, stage_z,
             send_x, recv_x, send_y, recv_y, send_z, recv_z):
        mx = lax.axis_index("x")
        my = lax.axis_index("y")
        mz = lax.axis_index("z")
        has_zlo = mz > 0
        has_zhi = mz < NZ - 1

        @pl.when(has_zlo)
        def _():
            stage_z[0] = u_ref[:, :, 0:1]

        @pl.when(has_zhi)
        def _():
            stage_z[1] = u_ref[:, :, S - 1:S]

        barrier_sem = pltpu.get_barrier_semaphore()
        pl.semaphore_signal(barrier_sem, inc=1, device_id=(1 - mx, my, mz),
                            device_id_type=pl.DeviceIdType.MESH)
        pl.semaphore_signal(barrier_sem, inc=1, device_id=(mx, 1 - my, mz),
                            device_id_type=pl.DeviceIdType.MESH)

        @pl.when(has_zlo)
        def _():
            pl.semaphore_signal(barrier_sem, inc=1, device_id=(mx, my, mz - 1),
                                device_id_type=pl.DeviceIdType.MESH)

        @pl.when(has_zhi)
        def _():
            pl.semaphore_signal(barrier_sem, inc=1, device_id=(mx, my, mz + 1),
                                device_id_type=pl.DeviceIdType.MESH)

        n_nbrs = 2 + has_zlo.astype(jnp.int32) + has_zhi.astype(jnp.int32)
        pl.semaphore_wait(barrier_sem, n_nbrs)

        slot_x = mx
        src_plane_x = (1 - mx) * (S - 1)
        rdma_x = pltpu.make_async_remote_copy(
            src_ref=u_ref.at[pl.ds(src_plane_x, 1), :, :],
            dst_ref=halo_x.at[slot_x],
            send_sem=send_x.at[slot_x],
            recv_sem=recv_x.at[slot_x],
            device_id=(1 - mx, my, mz),
            device_id_type=pl.DeviceIdType.MESH,
        )
        rdma_x.start()

        slot_y = my
        src_plane_y = (1 - my) * (S - 1)
        rdma_y = pltpu.make_async_remote_copy(
            src_ref=u_ref.at[:, pl.ds(src_plane_y, 1), :],
            dst_ref=halo_y.at[slot_y],
            send_sem=send_y.at[slot_y],
            recv_sem=recv_y.at[slot_y],
            device_id=(mx, 1 - my, mz),
            device_id_type=pl.DeviceIdType.MESH,
        )
        rdma_y.start()

        def z_send(slot, dz):
            return pltpu.make_async_remote_copy(
                src_ref=stage_z.at[slot],
                dst_ref=halo_z.at[1 - slot],
                send_sem=send_z.at[slot],
                recv_sem=recv_z.at[1 - slot],
                device_id=(mx, my, mz + dz),
                device_id_type=pl.DeviceIdType.MESH,
            )

        @pl.when(has_zlo)
        def _():
            z_send(0, -1).start()

        @pl.when(has_zhi)
        def _():
            z_send(1, 1).start()

        @pl.when(jnp.logical_not(has_zlo))
        def _():
            halo_z[0] = jnp.zeros((S, S, 1), jnp.float32)

        @pl.when(jnp.logical_not(has_zhi))
        def _():
            halo_z[1] = jnp.zeros((S, S, 1), jnp.float32)

        def recv_wait(halo, sends, recvs, slot, shape):
            pltpu.make_async_remote_copy(
                src_ref=halo.at[slot], dst_ref=halo.at[slot],
                send_sem=sends.at[slot], recv_sem=recvs.at[slot],
                device_id=(mx, my, mz),
                device_id_type=pl.DeviceIdType.MESH,
            ).wait_recv()

        recv_wait(halo_x, send_x, recv_x, 1 - mx, (1, S, S))
        recv_wait(halo_y, send_y, recv_y, 1 - my, (S, 1, S))

        @pl.when(has_zlo)
        def _():
            recv_wait(halo_z, send_z, recv_z, 0, (S, S, 1))

        @pl.when(has_zhi)
        def _():
            recv_wait(halo_z, send_z, recv_z, 1, (S, S, 1))

        rdma_x.wait_send()
        rdma_y.wait_send()

        @pl.when(has_zlo)
        def _():
            z_send(0, -1).wait_send()

        @pl.when(has_zhi)
        def _():
            z_send(1, 1).wait_send()

        uval = u_ref[...]
        xm = jnp.concatenate([halo_x[0], uval[:S - 1]], axis=0)
        xp = jnp.concatenate([uval[1:], halo_x[1]], axis=0)
        ym = jnp.concatenate([halo_y[0], uval[:, :S - 1, :]], axis=1)
        yp = jnp.concatenate([uval[:, 1:, :], halo_y[1]], axis=1)
        zm = jnp.concatenate([halo_z[0], uval[:, :, :S - 1]], axis=2)
        zp = jnp.concatenate([uval[:, :, 1:], halo_z[1]], axis=2)
        v = xm + xp + ym + yp + zm + zp - 6.0 * uval

        ix = lax.broadcasted_iota(jnp.int32, (S, S, S), 0) + mx * S
        iy = lax.broadcasted_iota(jnp.int32, (S, S, S), 1) + my * S
        iz = lax.broadcasted_iota(jnp.int32, (S, S, S), 2) + mz * S
        interior = (
            (ix > 0) & (ix < NX * S - 1)
            & (iy > 0) & (iy < NY * S - 1)
            & (iz > 0) & (iz < NZ * S - 1)
        )
        out_ref[...] = jnp.where(interior, v, 0.0)

    return pl.pallas_call(
        body,
        out_shape=jax.ShapeDtypeStruct((S, S, S), jnp.float32),
        in_specs=[pl.BlockSpec(memory_space=pltpu.VMEM)],
        out_specs=pl.BlockSpec(memory_space=pltpu.VMEM),
        scratch_shapes=[
            pltpu.VMEM((2, 1, S, S), jnp.float32),
            pltpu.VMEM((2, S, 1, S), jnp.float32),
            pltpu.VMEM((2, S, S, 1), jnp.float32),
            pltpu.VMEM((2, S, S, 1), jnp.float32),
            pltpu.SemaphoreType.DMA((2,)),
            pltpu.SemaphoreType.DMA((2,)),
            pltpu.SemaphoreType.DMA((2,)),
            pltpu.SemaphoreType.DMA((2,)),
            pltpu.SemaphoreType.DMA((2,)),
            pltpu.SemaphoreType.DMA((2,)),
        ],
        compiler_params=pltpu.CompilerParams(collective_id=0),
    )(u)


# device time: 12589 ns/iter; 1.0609x vs baseline; 1.0609x over previous
import jax
import jax.numpy as jnp
from jax import lax
from jax.experimental import pallas as pl
from jax.experimental.pallas import tpu as pltpu

NX, NY, NZ = 2, 2, 4
S = 32


def kernel(u):
    def body(u_ref, out_ref, halo_x, halo_y, halo_z, stage_z,
             send_x, recv_x, send_y, recv_y, send_z, recv_z):
        mx = lax.axis_index("x")
        my = lax.axis_index("y")
        mz = lax.axis_index("z")
        has_zlo = mz > 0
        has_zhi = mz < NZ - 1

        barrier_sem = pltpu.get_barrier_semaphore()
        pl.semaphore_signal(barrier_sem, inc=1, device_id=(1 - mx, my, mz),
                            device_id_type=pl.DeviceIdType.MESH)
        pl.semaphore_signal(barrier_sem, inc=1, device_id=(mx, 1 - my, mz),
                            device_id_type=pl.DeviceIdType.MESH)

        @pl.when(has_zlo)
        def _():
            pl.semaphore_signal(barrier_sem, inc=1, device_id=(mx, my, mz - 1),
                                device_id_type=pl.DeviceIdType.MESH)

        @pl.when(has_zhi)
        def _():
            pl.semaphore_signal(barrier_sem, inc=1, device_id=(mx, my, mz + 1),
                                device_id_type=pl.DeviceIdType.MESH)

        @pl.when(has_zlo)
        def _():
            stage_z[0] = u_ref[:, :, 0:1]

        @pl.when(has_zhi)
        def _():
            stage_z[1] = u_ref[:, :, S - 1:S]

        n_nbrs = 2 + has_zlo.astype(jnp.int32) + has_zhi.astype(jnp.int32)
        pl.semaphore_wait(barrier_sem, n_nbrs)

        slot_x = mx
        src_plane_x = (1 - mx) * (S - 1)
        rdma_x = pltpu.make_async_remote_copy(
            src_ref=u_ref.at[pl.ds(src_plane_x, 1), :, :],
            dst_ref=halo_x.at[slot_x],
            send_sem=send_x.at[slot_x],
            recv_sem=recv_x.at[slot_x],
            device_id=(1 - mx, my, mz),
            device_id_type=pl.DeviceIdType.MESH,
        )
        rdma_x.start()

        slot_y = my
        src_plane_y = (1 - my) * (S - 1)
        rdma_y = pltpu.make_async_remote_copy(
            src_ref=u_ref.at[:, pl.ds(src_plane_y, 1), :],
            dst_ref=halo_y.at[slot_y],
            send_sem=send_y.at[slot_y],
            recv_sem=recv_y.at[slot_y],
            device_id=(mx, 1 - my, mz),
            device_id_type=pl.DeviceIdType.MESH,
        )
        rdma_y.start()

        def z_send(slot, dz):
            return pltpu.make_async_remote_copy(
                src_ref=stage_z.at[slot],
                dst_ref=halo_z.at[1 - slot],
                send_sem=send_z.at[slot],
                recv_sem=recv_z.at[1 - slot],
                device_id=(mx, my, mz + dz),
                device_id_type=pl.DeviceIdType.MESH,
            )

        @pl.when(has_zlo)
        def _():
            z_send(0, -1).start()

        @pl.when(has_zhi)
        def _():
            z_send(1, 1).start()

        uval = u_ref[...]
        zx = jnp.zeros((1, S, S), jnp.float32)
        zy = jnp.zeros((S, 1, S), jnp.float32)
        zz = jnp.zeros((S, S, 1), jnp.float32)
        core = (
            jnp.concatenate([zx, uval[:S - 1]], axis=0)
            + jnp.concatenate([uval[1:], zx], axis=0)
            + jnp.concatenate([zy, uval[:, :S - 1, :]], axis=1)
            + jnp.concatenate([uval[:, 1:, :], zy], axis=1)
            + jnp.concatenate([zz, uval[:, :, :S - 1]], axis=2)
            + jnp.concatenate([uval[:, :, 1:], zz], axis=2)
            - 6.0 * uval
        )
        ix = lax.broadcasted_iota(jnp.int32, (S, S, S), 0) + mx * S
        iy = lax.broadcasted_iota(jnp.int32, (S, S, S), 1) + my * S
        iz = lax.broadcasted_iota(jnp.int32, (S, S, S), 2) + mz * S
        interior = (
            (ix > 0) & (ix < NX * S - 1)
            & (iy > 0) & (iy < NY * S - 1)
            & (iz > 0) & (iz < NZ * S - 1)
        )
        out_ref[...] = jnp.where(interior, core, 0.0)

        def recv_wait(halo, sends, recvs, slot):
            pltpu.make_async_remote_copy(
                src_ref=halo.at[slot], dst_ref=halo.at[slot],
                send_sem=sends.at[slot], recv_sem=recvs.at[slot],
                device_id=(mx, my, mz),
                device_id_type=pl.DeviceIdType.MESH,
            ).wait_recv()

        recv_wait(halo_x, send_x, recv_x, 1 - mx)
        recv_wait(halo_y, send_y, recv_y, 1 - my)

        @pl.when(has_zlo)
        def _():
            recv_wait(halo_z, send_z, recv_z, 0)

        @pl.when(has_zhi)
        def _():
            recv_wait(halo_z, send_z, recv_z, 1)

        iy_f = lax.broadcasted_iota(jnp.int32, (1, S, S), 1) + my * S
        iz_f = lax.broadcasted_iota(jnp.int32, (1, S, S), 2) + mz * S
        m_yz = (iy_f > 0) & (iy_f < NY * S - 1) & (iz_f > 0) & (iz_f < NZ * S - 1)

        ix_f = lax.broadcasted_iota(jnp.int32, (S, 1, S), 0) + mx * S
        iz_f2 = lax.broadcasted_iota(jnp.int32, (S, 1, S), 2) + mz * S
        m_xz = (ix_f > 0) & (ix_f < NX * S - 1) & (iz_f2 > 0) & (iz_f2 < NZ * S - 1)

        ix_f3 = lax.broadcasted_iota(jnp.int32, (S, S, 1), 0) + mx * S
        iy_f3 = lax.broadcasted_iota(jnp.int32, (S, S, 1), 1) + my * S
        m_xy = (ix_f3 > 0) & (ix_f3 < NX * S - 1) & (iy_f3 > 0) & (iy_f3 < NY * S - 1)

        @pl.when(mx > 0)
        def _():
            out_ref[0:1, :, :] = out_ref[0:1, :, :] + jnp.where(
                m_yz, halo_x[0], 0.0)

        @pl.when(mx < NX - 1)
        def _():
            out_ref[S - 1:S, :, :] = out_ref[S - 1:S, :, :] + jnp.where(
                m_yz, halo_x[1], 0.0)

        @pl.when(my > 0)
        def _():
            out_ref[:, 0:1, :] = out_ref[:, 0:1, :] + jnp.where(
                m_xz, halo_y[0], 0.0)

        @pl.when(my < NY - 1)
        def _():
            out_ref[:, S - 1:S, :] = out_ref[:, S - 1:S, :] + jnp.where(
                m_xz, halo_y[1], 0.0)

        @pl.when(has_zlo)
        def _():
            out_ref[:, :, 0:1] = out_ref[:, :, 0:1] + jnp.where(
                m_xy, halo_z[0], 0.0)

        @pl.when(has_zhi)
        def _():
            out_ref[:, :, S - 1:S] = out_ref[:, :, S - 1:S] + jnp.where(
                m_xy, halo_z[1], 0.0)

        rdma_x.wait_send()
        rdma_y.wait_send()

        @pl.when(has_zlo)
        def _():
            z_send(0, -1).wait_send()

        @pl.when(has_zhi)
        def _():
            z_send(1, 1).wait_send()

    return pl.pallas_call(
        body,
        out_shape=jax.ShapeDtypeStruct((S, S, S), jnp.float32),
        in_specs=[pl.BlockSpec(memory_space=pltpu.VMEM)],
        out_specs=pl.BlockSpec(memory_space=pltpu.VMEM),
        scratch_shapes=[
            pltpu.VMEM((2, 1, S, S), jnp.float32),
            pltpu.VMEM((2, S, 1, S), jnp.float32),
            pltpu.VMEM((2, S, S, 1), jnp.float32),
            pltpu.VMEM((2, S, S, 1), jnp.float32),
            pltpu.SemaphoreType.DMA((2,)),
            pltpu.SemaphoreType.DMA((2,)),
            pltpu.SemaphoreType.DMA((2,)),
            pltpu.SemaphoreType.DMA((2,)),
            pltpu.SemaphoreType.DMA((2,)),
            pltpu.SemaphoreType.DMA((2,)),
        ],
        compiler_params=pltpu.CompilerParams(collective_id=0),
    )(u)


# device time: 12474 ns/iter; 1.0707x vs baseline; 1.0092x over previous
import jax
import jax.numpy as jnp
from jax import lax
from jax.experimental import pallas as pl
from jax.experimental.pallas import tpu as pltpu

NX, NY, NZ = 2, 2, 4
S = 32


def kernel(u):
    def body(u_ref, out_ref, halo_x, halo_y, halo_z, stage_z,
             send_x, recv_x, send_y, recv_y, send_z, recv_z):
        mx = lax.axis_index("x")
        my = lax.axis_index("y")
        mz = lax.axis_index("z")
        has_zlo = mz > 0
        has_zhi = mz < NZ - 1

        barrier_sem = pltpu.get_barrier_semaphore()
        pl.semaphore_signal(barrier_sem, inc=1, device_id=(1 - mx, my, mz),
                            device_id_type=pl.DeviceIdType.MESH)
        pl.semaphore_signal(barrier_sem, inc=1, device_id=(mx, 1 - my, mz),
                            device_id_type=pl.DeviceIdType.MESH)

        @pl.when(has_zlo)
        def _():
            pl.semaphore_signal(barrier_sem, inc=1, device_id=(mx, my, mz - 1),
                                device_id_type=pl.DeviceIdType.MESH)

        @pl.when(has_zhi)
        def _():
            pl.semaphore_signal(barrier_sem, inc=1, device_id=(mx, my, mz + 1),
                                device_id_type=pl.DeviceIdType.MESH)

        @pl.when(has_zlo)
        def _():
            stage_z[0] = u_ref[:, :, 0:1]

        @pl.when(has_zhi)
        def _():
            stage_z[1] = u_ref[:, :, S - 1:S]

        n_nbrs = 2 + has_zlo.astype(jnp.int32) + has_zhi.astype(jnp.int32)
        pl.semaphore_wait(barrier_sem, n_nbrs)

        slot_x = mx
        src_plane_x = (1 - mx) * (S - 1)
        rdma_x = pltpu.make_async_remote_copy(
            src_ref=u_ref.at[pl.ds(src_plane_x, 1), :, :],
            dst_ref=halo_x.at[slot_x],
            send_sem=send_x.at[slot_x],
            recv_sem=recv_x.at[slot_x],
            device_id=(1 - mx, my, mz),
            device_id_type=pl.DeviceIdType.MESH,
        )
        rdma_x.start()

        slot_y = my
        src_plane_y = (1 - my) * (S - 1)
        rdma_y = pltpu.make_async_remote_copy(
            src_ref=u_ref.at[:, pl.ds(src_plane_y, 1), :],
            dst_ref=halo_y.at[slot_y],
            send_sem=send_y.at[slot_y],
            recv_sem=recv_y.at[slot_y],
            device_id=(mx, 1 - my, mz),
            device_id_type=pl.DeviceIdType.MESH,
        )
        rdma_y.start()

        def z_send(slot, dz):
            return pltpu.make_async_remote_copy(
                src_ref=stage_z.at[slot],
                dst_ref=halo_z.at[1 - slot],
                send_sem=send_z.at[slot],
                recv_sem=recv_z.at[1 - slot],
                device_id=(mx, my, mz + dz),
                device_id_type=pl.DeviceIdType.MESH,
            )

        @pl.when(has_zlo)
        def _():
            z_send(0, -1).start()

        @pl.when(has_zhi)
        def _():
            z_send(1, 1).start()

        uval = u_ref[...]
        zx = jnp.zeros((1, S, S), jnp.float32)
        zy = jnp.zeros((S, 1, S), jnp.float32)
        zz = jnp.zeros((S, S, 1), jnp.float32)
        core = (
            jnp.concatenate([zx, uval[:S - 1]], axis=0)
            + jnp.concatenate([uval[1:], zx], axis=0)
            + jnp.concatenate([zy, uval[:, :S - 1, :]], axis=1)
            + jnp.concatenate([uval[:, 1:, :], zy], axis=1)
            + jnp.concatenate([zz, uval[:, :, :S - 1]], axis=2)
            + jnp.concatenate([uval[:, :, 1:], zz], axis=2)
            - 6.0 * uval
        )
        ix = lax.broadcasted_iota(jnp.int32, (S, S, S), 0) + mx * S
        iy = lax.broadcasted_iota(jnp.int32, (S, S, S), 1) + my * S
        iz = lax.broadcasted_iota(jnp.int32, (S, S, S), 2) + mz * S
        interior = (
            (ix > 0) & (ix < NX * S - 1)
            & (iy > 0) & (iy < NY * S - 1)
            & (iz > 0) & (iz < NZ * S - 1)
        )
        out_ref[...] = jnp.where(interior, core, 0.0)

        iy_f = lax.broadcasted_iota(jnp.int32, (1, S, S), 1) + my * S
        iz_f = lax.broadcasted_iota(jnp.int32, (1, S, S), 2) + mz * S
        m_yz = (iy_f > 0) & (iy_f < NY * S - 1) & (iz_f > 0) & (iz_f < NZ * S - 1)

        ix_f = lax.broadcasted_iota(jnp.int32, (S, 1, S), 0) + mx * S
        iz_f2 = lax.broadcasted_iota(jnp.int32, (S, 1, S), 2) + mz * S
        m_xz = (ix_f > 0) & (ix_f < NX * S - 1) & (iz_f2 > 0) & (iz_f2 < NZ * S - 1)

        ix_f3 = lax.broadcasted_iota(jnp.int32, (S, S, 1), 0) + mx * S
        iy_f3 = lax.broadcasted_iota(jnp.int32, (S, S, 1), 1) + my * S
        m_xy = (ix_f3 > 0) & (ix_f3 < NX * S - 1) & (iy_f3 > 0) & (iy_f3 < NY * S - 1)

        def recv_wait(halo, sends, recvs, slot):
            pltpu.make_async_remote_copy(
                src_ref=halo.at[slot], dst_ref=halo.at[slot],
                send_sem=sends.at[slot], recv_sem=recvs.at[slot],
                device_id=(mx, my, mz),
                device_id_type=pl.DeviceIdType.MESH,
            ).wait_recv()

        recv_wait(halo_x, send_x, recv_x, 1 - mx)

        @pl.when(mx > 0)
        def _():
            out_ref[0:1, :, :] = out_ref[0:1, :, :] + jnp.where(
                m_yz, halo_x[0], 0.0)

        @pl.when(mx < NX - 1)
        def _():
            out_ref[S - 1:S, :, :] = out_ref[S - 1:S, :, :] + jnp.where(
                m_yz, halo_x[1], 0.0)

        recv_wait(halo_y, send_y, recv_y, 1 - my)

        @pl.when(my > 0)
        def _():
            out_ref[:, 0:1, :] = out_ref[:, 0:1, :] + jnp.where(
                m_xz, halo_y[0], 0.0)

        @pl.when(my < NY - 1)
        def _():
            out_ref[:, S - 1:S, :] = out_ref[:, S - 1:S, :] + jnp.where(
                m_xz, halo_y[1], 0.0)

        @pl.when(has_zlo)
        def _():
            recv_wait(halo_z, send_z, recv_z, 0)
            out_ref[:, :, 0:1] = out_ref[:, :, 0:1] + jnp.where(
                m_xy, halo_z[0], 0.0)

        @pl.when(has_zhi)
        def _():
            recv_wait(halo_z, send_z, recv_z, 1)
            out_ref[:, :, S - 1:S] = out_ref[:, :, S - 1:S] + jnp.where(
                m_xy, halo_z[1], 0.0)

        rdma_x.wait_send()
        rdma_y.wait_send()

        @pl.when(has_zlo)
        def _():
            z_send(0, -1).wait_send()

        @pl.when(has_zhi)
        def _():
            z_send(1, 1).wait_send()

    return pl.pallas_call(
        body,
        out_shape=jax.ShapeDtypeStruct((S, S, S), jnp.float32),
        in_specs=[pl.BlockSpec(memory_space=pltpu.VMEM)],
        out_specs=pl.BlockSpec(memory_space=pltpu.VMEM),
        scratch_shapes=[
            pltpu.VMEM((2, 1, S, S), jnp.float32),
            pltpu.VMEM((2, S, 1, S), jnp.float32),
            pltpu.VMEM((2, S, S, 1), jnp.float32),
            pltpu.VMEM((2, S, S, 1), jnp.float32),
            pltpu.SemaphoreType.DMA((2,)),
            pltpu.SemaphoreType.DMA((2,)),
            pltpu.SemaphoreType.DMA((2,)),
            pltpu.SemaphoreType.DMA((2,)),
            pltpu.SemaphoreType.DMA((2,)),
            pltpu.SemaphoreType.DMA((2,)),
        ],
        compiler_params=pltpu.CompilerParams(collective_id=0),
    )(u)
